# baseline (speedup 1.0000x reference)
import jax
import jax.numpy as jnp
from jax.experimental import pallas as pl
from jax.experimental.pallas import tpu as pltpu

_TB = 1024
_S = 10


def _body(a_ref, b_ref, c_ref, ws0_ref, wn0_ref, ws1_ref, wn1_ref,
          out_ref, h0a_ref, accb_ref, m1a_ref, m1b_ref):
    s = pl.program_id(1)
    f32 = jnp.float32
    relu = jax.nn.relu
    b = b_ref[...]
    c = c_ref[...]
    ws0 = ws0_ref[...]
    wn0 = wn0_ref[...]
    bs = relu(jnp.dot(b, ws0, preferred_element_type=f32))
    cs = relu(jnp.dot(c, wn0, preferred_element_type=f32))

    @pl.when(s == 0)
    def _():
        a = a_ref[...]
        h0a_ref[...] = relu(jnp.dot(a, ws0, preferred_element_type=f32))
        accb_ref[...] = b
        m1a_ref[...] = bs
        m1b_ref[...] = cs

    @pl.when(s > 0)
    def _():
        accb_ref[...] += b
        m1a_ref[...] += bs
        m1b_ref[...] += cs

    @pl.when(s == _S - 1)
    def _():
        inv = f32(1.0 / _S)
        mean_b = accb_ref[...] * inv
        h0b = relu(jnp.dot(mean_b, wn0, preferred_element_type=f32))
        h0a = h0a_ref[...]
        m1a = m1a_ref[...] * inv
        m1b = m1b_ref[...] * inv
        ws1 = ws1_ref[...]
        wn1 = wn1_ref[...]
        out_ref[:, :128] = (jnp.dot(h0a, ws1[:128], preferred_element_type=f32)
                            + jnp.dot(h0b, ws1[128:], preferred_element_type=f32))
        out_ref[:, 128:] = (jnp.dot(m1a, wn1[:128], preferred_element_type=f32)
                            + jnp.dot(m1b, wn1[128:], preferred_element_type=f32))


def kernel(input_features, W_self_0, W_neigh_0, W_self_1, W_neigh_1):
    n, _, f = input_features.shape
    d1 = W_self_0.shape[1]
    d2 = W_self_1.shape[1]
    tb = _TB
    grid = (n // tb, _S)
    x2 = input_features.reshape(n, input_features.shape[1] * f)
    a_spec = pl.BlockSpec((tb, f), lambda i, s: (i, 0))
    b_spec = pl.BlockSpec((tb, f), lambda i, s: (i, 1 + s))
    c_spec = pl.BlockSpec((tb, f), lambda i, s: (i, 1 + _S + s))
    w0_spec = pl.BlockSpec((f, d1), lambda i, s: (0, 0))
    w1_spec = pl.BlockSpec((2 * d1, d2), lambda i, s: (0, 0))
    out_spec = pl.BlockSpec((tb, 2 * d2), lambda i, s: (i, 0))
    return pl.pallas_call(
        _body,
        grid=grid,
        in_specs=[a_spec, b_spec, c_spec, w0_spec, w0_spec, w1_spec, w1_spec],
        out_specs=out_spec,
        out_shape=jax.ShapeDtypeStruct((n, 2 * d2), jnp.float32),
        scratch_shapes=[pltpu.VMEM((tb, d1), jnp.float32) for _ in range(4)],
    )(x2, x2, x2, W_self_0, W_neigh_0, W_self_1, W_neigh_1)


# speedup vs baseline: 1.2427x; 1.2427x over previous
import jax
import jax.numpy as jnp
from jax.experimental import pallas as pl
from jax.experimental.pallas import tpu as pltpu

_TB = 1024
_S = 10


def _dot(x, w):
    return jax.lax.dot_general(
        x.astype(jnp.bfloat16), w.astype(jnp.bfloat16),
        (((1,), (0,)), ((), ())),
        preferred_element_type=jnp.float32)


def _body(x_hbm, ws0_ref, wn0_ref, ws1_ref, wn1_ref, out_ref,
          abuf, bbuf, cbuf, accb_ref, m1a_ref, m1b_ref, h0a_ref,
          bsem, csem, asem):
    i = pl.program_id(0)
    s = pl.program_id(1)
    nt = pl.num_programs(0)
    f32 = jnp.float32
    relu = jax.nn.relu

    def issue(it, st, par):
        row0 = it * _TB
        pltpu.make_async_copy(
            x_hbm.at[pl.ds(row0, _TB), 1 + st], bbuf.at[par], bsem.at[par]).start()
        pltpu.make_async_copy(
            x_hbm.at[pl.ds(row0, _TB), 1 + _S + st], cbuf.at[par], csem.at[par]).start()

    @pl.when((i == 0) & (s == 0))
    def _():
        issue(i, s, 0)
        pltpu.make_async_copy(x_hbm.at[pl.ds(0, _TB), 0], abuf, asem).start()

    @pl.when(jnp.logical_not((i == nt - 1) & (s == _S - 1)))
    def _():
        wrap = s == _S - 1
        s2 = jnp.where(wrap, 0, s + 1)
        i2 = jnp.where(wrap, i + 1, i)
        issue(i2, s2, s2 % 2)

    @pl.when(s == _S - 1)
    def _():
        @pl.when(i < nt - 1)
        def _():
            pltpu.make_async_copy(
                x_hbm.at[pl.ds((i + 1) * _TB, _TB), 0], abuf, asem).start()

    par = s % 2
    pltpu.make_async_copy(x_hbm.at[pl.ds(0, _TB), 0], bbuf.at[par], bsem.at[par]).wait()
    pltpu.make_async_copy(x_hbm.at[pl.ds(0, _TB), 0], cbuf.at[par], csem.at[par]).wait()
    b = bbuf[par]
    c = cbuf[par]
    ws0 = ws0_ref[...]
    wn0 = wn0_ref[...]
    bs = relu(_dot(b, ws0))
    cs = relu(_dot(c, wn0))

    @pl.when(s == 0)
    def _():
        pltpu.make_async_copy(x_hbm.at[pl.ds(0, _TB), 0], abuf, asem).wait()
        h0a_ref[...] = relu(_dot(abuf[...], ws0))
        accb_ref[...] = b
        m1a_ref[...] = bs
        m1b_ref[...] = cs

    @pl.when(s > 0)
    def _():
        accb_ref[...] += b
        m1a_ref[...] += bs
        m1b_ref[...] += cs

    @pl.when(s == _S - 1)
    def _():
        inv = f32(1.0 / _S)
        h0b = relu(_dot(accb_ref[...] * inv, wn0))
        h0a = h0a_ref[...]
        m1a = m1a_ref[...] * inv
        m1b = m1b_ref[...] * inv
        ws1 = ws1_ref[...]
        wn1 = wn1_ref[...]
        d1 = ws0.shape[1]
        out_ref[:, :d1] = _dot(h0a, ws1[:d1]) + _dot(h0b, ws1[d1:])
        out_ref[:, d1:] = _dot(m1a, wn1[:d1]) + _dot(m1b, wn1[d1:])


def kernel(input_features, W_self_0, W_neigh_0, W_self_1, W_neigh_1):
    n, slots, f = input_features.shape
    d1 = W_self_0.shape[1]
    d2 = W_self_1.shape[1]
    tb = _TB
    grid = (n // tb, _S)
    return pl.pallas_call(
        _body,
        grid=grid,
        in_specs=[
            pl.BlockSpec(memory_space=pl.ANY),
            pl.BlockSpec((f, d1), lambda i, s: (0, 0)),
            pl.BlockSpec((f, d1), lambda i, s: (0, 0)),
            pl.BlockSpec((2 * d1, d2), lambda i, s: (0, 0)),
            pl.BlockSpec((2 * d1, d2), lambda i, s: (0, 0)),
        ],
        out_specs=pl.BlockSpec((tb, 2 * d2), lambda i, s: (i, 0)),
        out_shape=jax.ShapeDtypeStruct((n, 2 * d2), jnp.float32),
        scratch_shapes=[
            pltpu.VMEM((tb, f), jnp.float32),
            pltpu.VMEM((2, tb, f), jnp.float32),
            pltpu.VMEM((2, tb, f), jnp.float32),
            pltpu.VMEM((tb, d1), jnp.float32),
            pltpu.VMEM((tb, d1), jnp.float32),
            pltpu.VMEM((tb, d1), jnp.float32),
            pltpu.VMEM((tb, d1), jnp.float32),
            pltpu.SemaphoreType.DMA((2,)),
            pltpu.SemaphoreType.DMA((2,)),
            pltpu.SemaphoreType.DMA,
        ],
    )(input_features, W_self_0, W_neigh_0, W_self_1, W_neigh_1)


# speedup vs baseline: 1.7254x; 1.3885x over previous
import jax
import jax.numpy as jnp
from jax.experimental import pallas as pl
from jax.experimental.pallas import tpu as pltpu

_TB = 1024
_S = 10
_NSLOT = 1 + 2 * _S


def _dot(x, w):
    return jax.lax.dot_general(
        x.astype(jnp.bfloat16), w,
        (((1,), (0,)), ((), ())),
        preferred_element_type=jnp.float32)


def _body(x_hbm, ws0_ref, wn0_ref, ws1_ref, wn1_ref, out_ref, buf, sem):
    i = pl.program_id(0)
    nt = pl.num_programs(0) - 1
    f32 = jnp.float32
    relu = jax.nn.relu

    @pl.when(i < nt)
    def _():
        par = i % 2
        row0 = i * _TB
        for s in range(_NSLOT):
            pltpu.make_async_copy(
                x_hbm.at[pl.ds(row0, _TB), s], buf.at[par, s], sem.at[par, s]).start()

    @pl.when(i > 0)
    def _():
        par = (i - 1) % 2
        for s in range(_NSLOT):
            pltpu.make_async_copy(
                x_hbm.at[pl.ds(0, _TB), s], buf.at[par, s], sem.at[par, s]).wait()
        ws0 = ws0_ref[...].astype(jnp.bfloat16)
        wn0 = wn0_ref[...].astype(jnp.bfloat16)
        inv = f32(1.0 / _S)

        h0a = relu(_dot(buf[par, 0], ws0))
        accb = buf[par, 1]
        m1a = relu(_dot(buf[par, 1], ws0))
        m1b = relu(_dot(buf[par, 1 + _S], wn0))
        for s in range(2, _S + 1):
            accb = accb + buf[par, s]
            m1a = m1a + relu(_dot(buf[par, s], ws0))
            m1b = m1b + relu(_dot(buf[par, s + _S], wn0))
        h0b = relu(_dot(accb * inv, wn0))
        m1a = m1a * inv
        m1b = m1b * inv

        ws1 = ws1_ref[...].astype(jnp.bfloat16)
        wn1 = wn1_ref[...].astype(jnp.bfloat16)
        d1 = ws0.shape[1]
        out_ref[:, :d1] = _dot(h0a, ws1[:d1]) + _dot(h0b, ws1[d1:])
        out_ref[:, d1:] = _dot(m1a, wn1[:d1]) + _dot(m1b, wn1[d1:])


def kernel(input_features, W_self_0, W_neigh_0, W_self_1, W_neigh_1):
    n, slots, f = input_features.shape
    d1 = W_self_0.shape[1]
    d2 = W_self_1.shape[1]
    tb = _TB
    nt = n // tb
    return pl.pallas_call(
        _body,
        grid=(nt + 1,),
        in_specs=[
            pl.BlockSpec(memory_space=pl.ANY),
            pl.BlockSpec((f, d1), lambda i: (0, 0)),
            pl.BlockSpec((f, d1), lambda i: (0, 0)),
            pl.BlockSpec((2 * d1, d2), lambda i: (0, 0)),
            pl.BlockSpec((2 * d1, d2), lambda i: (0, 0)),
        ],
        out_specs=pl.BlockSpec(
            (tb, 2 * d2), lambda i: (jnp.maximum(i - 1, 0), 0)),
        out_shape=jax.ShapeDtypeStruct((n, 2 * d2), jnp.float32),
        scratch_shapes=[
            pltpu.VMEM((2, _NSLOT, tb, f), jnp.float32),
            pltpu.SemaphoreType.DMA((2, _NSLOT)),
        ],
    )(input_features, W_self_0, W_neigh_0, W_self_1, W_neigh_1)
